# Initial kernel scaffold; baseline (speedup 1.0000x reference)
import jax
import jax.numpy as jnp
from jax.experimental import pallas as pl


def kernel(x, edge_index, edge_attr, W1, b1, Wm1, bm1, W2, b2, Wm2, bm2, Wc1, bc1, Wc2, bc2, Wc3, bc3, Wc4, bc4, Wo, bo):
    raise NotImplementedError("write your pallas kernel here")


# speedup vs baseline: 3.3620x; 3.3620x over previous
import functools

import jax
import jax.numpy as jnp
from jax import lax
from jax.experimental import pallas as pl
from jax.experimental.pallas import tpu as pltpu
from jax.experimental.pallas import tpu_sc as plsc

N = 10000
E = 320000
D = 128
OUT = 10

NW = 32
NC = 2
NPW = 320
NPAD = NW * NPW
W = 256
NEG_INF = float("-inf")

_mesh = plsc.VectorSubcoreMesh(core_axis_name="c", subcore_axis_name="s")


def _wid():
    return lax.axis_index("s") * NC + lax.axis_index("c")


def _edgemax_body(b_hbm, src_hbm, dst_hbm, ws_hbm, m_hbm,
                  ws_v, idx_v, dst_v, rows_v, mloc, sem):
    wid = _wid()
    n0 = wid * NPW
    pltpu.sync_copy(ws_hbm, ws_v)
    e_lo = ws_v[pl.ds(wid, 16)][0]
    e_hi = ws_v[pl.ds(wid + 1, 16)][0]

    neg = jnp.full((16,), NEG_INF, jnp.float32)

    def ibody(r, _):
        for v in range(8):
            mloc[r, pl.ds(v * 16, 16)] = neg
        return 0

    lax.fori_loop(0, NPW, ibody, 0)

    a_lo = (e_lo // 8) * 8
    nwin = (e_hi - a_lo + W - 1) // W

    def wbody(k, _):
        start = a_lo + k * W
        base = jnp.minimum(start, E - W)
        lo_k = jnp.maximum(e_lo, start)
        pltpu.sync_copy(dst_hbm.at[pl.ds(base, W)], dst_v)
        for j in range(2):
            pltpu.sync_copy(src_hbm.at[pl.ds(base + j * 128, 128)], idx_v.at[j])
        cps = [pltpu.async_copy(b_hbm.at[idx_v.at[j]],
                                rows_v.at[pl.ds(j * 128, 128)], sem)
               for j in range(2)]
        for cp in cps:
            cp.wait()

        def gbody(g, _):
            off16 = g * 16
            dst16 = dst_v[pl.ds(off16, 16)]
            for lane in range(16):
                i = off16 + lane
                e = base + i
                d = dst16[lane]
                valid = jnp.logical_and(e >= lo_k, e < e_hi)
                r = d - n0

                @pl.when(valid)
                def _():
                    for v in range(8):
                        cur = mloc[r, pl.ds(v * 16, 16)]
                        g_ = rows_v[i, pl.ds(v * 16, 16)]
                        mloc[r, pl.ds(v * 16, 16)] = jnp.maximum(cur, g_)

            return 0

        lax.fori_loop(0, W // 16, gbody, 0)
        return 0

    lax.fori_loop(0, nwin, wbody, 0)
    pltpu.sync_copy(mloc, m_hbm.at[pl.ds(n0, NPW)])


_edgemax = pl.kernel(
    _edgemax_body,
    out_type=jax.ShapeDtypeStruct((NPAD, D), jnp.float32),
    mesh=_mesh,
    scratch_types=[
        pltpu.VMEM((48,), jnp.int32),
        pltpu.VMEM((2, 128), jnp.int32),
        pltpu.VMEM((W,), jnp.int32),
        pltpu.VMEM((W, D), jnp.float32),
        pltpu.VMEM((NPW, D), jnp.float32),
        pltpu.SemaphoreType.DMA,
    ],
)


def _emsum_body(c_hbm, b_hbm, oth_hbm, key_hbm, ea_hbm, w_hbm, ws_hbm, agg_hbm,
                ws_v, idx_v, key_v, ea_v, w_v, rows_v, bloc, aggloc, sem):
    wid = _wid()
    n0 = wid * NPW
    pltpu.sync_copy(ws_hbm, ws_v)
    pltpu.sync_copy(w_hbm, w_v)
    pltpu.sync_copy(b_hbm.at[pl.ds(n0, NPW)], bloc)
    e_lo = ws_v[pl.ds(wid, 16)][0]
    e_hi = ws_v[pl.ds(wid + 1, 16)][0]

    zero = jnp.zeros((16,), jnp.float32)

    def ibody(r, _):
        for v in range(8):
            aggloc[r, pl.ds(v * 16, 16)] = zero
        return 0

    lax.fori_loop(0, NPW, ibody, 0)

    wvec = [w_v[pl.ds(v * 16, 16)] for v in range(8)]

    a_lo = (e_lo // 8) * 8
    nwin = (e_hi - a_lo + W - 1) // W

    def wbody(k, _):
        start = a_lo + k * W
        base = jnp.minimum(start, E - W)
        lo_k = jnp.maximum(e_lo, start)
        pltpu.sync_copy(key_hbm.at[pl.ds(base, W)], key_v)
        pltpu.sync_copy(ea_hbm.at[pl.ds(base, W)], ea_v)
        for j in range(2):
            pltpu.sync_copy(oth_hbm.at[pl.ds(base + j * 128, 128)], idx_v.at[j])
        cps = [pltpu.async_copy(c_hbm.at[idx_v.at[j]],
                                rows_v.at[pl.ds(j * 128, 128)], sem)
               for j in range(2)]
        for cp in cps:
            cp.wait()

        def gbody(g, _):
            off16 = g * 16
            key16 = key_v[pl.ds(off16, 16)]
            ea16 = ea_v[pl.ds(off16, 16)]
            for lane in range(16):
                i = off16 + lane
                e = base + i
                kk = key16[lane]
                ea = ea16[lane]
                valid = jnp.logical_and(e >= lo_k, e < e_hi)
                r = kk - n0

                @pl.when(valid)
                def _():
                    for v in range(8):
                        c = rows_v[i, pl.ds(v * 16, 16)]
                        bb = bloc[r, pl.ds(v * 16, 16)]
                        val = jnp.maximum(c + bb + ea * wvec[v], 0.0)
                        aggloc[r, pl.ds(v * 16, 16)] = (
                            aggloc[r, pl.ds(v * 16, 16)] + val)

            return 0

        lax.fori_loop(0, W // 16, gbody, 0)
        return 0

    lax.fori_loop(0, nwin, wbody, 0)
    pltpu.sync_copy(aggloc, agg_hbm.at[pl.ds(n0, NPW)])


_emsum = pl.kernel(
    _emsum_body,
    out_type=jax.ShapeDtypeStruct((NPAD, D), jnp.float32),
    mesh=_mesh,
    scratch_types=[
        pltpu.VMEM((48,), jnp.int32),
        pltpu.VMEM((2, 128), jnp.int32),
        pltpu.VMEM((W,), jnp.int32),
        pltpu.VMEM((W,), jnp.float32),
        pltpu.VMEM((D,), jnp.float32),
        pltpu.VMEM((W, D), jnp.float32),
        pltpu.VMEM((NPW, D), jnp.float32),
        pltpu.VMEM((NPW, D), jnp.float32),
        pltpu.SemaphoreType.DMA,
    ],
)


def _dot(a, b):
    return jnp.dot(a, b, preferred_element_type=jnp.float32,
                   precision=jax.lax.Precision.HIGHEST)


def _tc_head_body(x_ref, w1_ref, b1_ref, wcat_ref, bcat_ref, h_ref, c_ref, b_ref):
    h = jnp.maximum(_dot(x_ref[...], w1_ref[...]) + b1_ref[...], 0.0)
    h_ref[...] = h
    cb = _dot(h, wcat_ref[...]) + bcat_ref[...]
    c_ref[...] = cb[:, :D]
    b_ref[...] = cb[:, D:]


def _tc_em_mid_body(h_ref_in, agg_ref, w2_ref, b2_ref, wcat_ref, bcat_ref,
                    h_ref, c_ref, b_ref):
    h1 = (h_ref_in[...] + agg_ref[...]) * 0.5
    h2 = jnp.maximum(_dot(h1, w2_ref[...]) + b2_ref[...], 0.0)
    h_ref[...] = h2
    cb = _dot(h2, wcat_ref[...]) + bcat_ref[...]
    c_ref[...] = cb[:, :D]
    b_ref[...] = cb[:, D:]


def _tc_em2ec_body(h_ref_in, agg_ref, wcat_ref, bcat_ref, c_ref, b_ref):
    h3 = (h_ref_in[...] + agg_ref[...]) * 0.5
    cb = _dot(h3, wcat_ref[...]) + bcat_ref[...]
    c_ref[...] = cb[:, :D]
    b_ref[...] = cb[:, D:]


def _tc_ec_mid_body(cprev_ref, m_ref, wcat_ref, bcat_ref, c_ref, b_ref):
    m = m_ref[...]
    h = jnp.where(m == NEG_INF, 0.0, jnp.maximum(cprev_ref[...] + m, 0.0))
    cb = _dot(h, wcat_ref[...]) + bcat_ref[...]
    c_ref[...] = cb[:, :D]
    b_ref[...] = cb[:, D:]


def _tc_out_body(cprev_ref, m_ref, wo_ref, bo_ref, y_ref):
    m = m_ref[...]
    h = jnp.where(m == NEG_INF, 0.0, jnp.maximum(cprev_ref[...] + m, 0.0))
    y_ref[...] = _dot(h, wo_ref[...]) + bo_ref[...]


_f32 = jnp.float32
_nd = jax.ShapeDtypeStruct((NPAD, D), _f32)

_tc_head = pl.pallas_call(_tc_head_body, out_shape=[_nd, _nd, _nd])
_tc_em_mid = pl.pallas_call(_tc_em_mid_body, out_shape=[_nd, _nd, _nd])
_tc_em2ec = pl.pallas_call(_tc_em2ec_body, out_shape=[_nd, _nd])
_tc_ec_mid = pl.pallas_call(_tc_ec_mid_body, out_shape=[_nd, _nd])
_tc_out = pl.pallas_call(_tc_out_body, out_shape=jax.ShapeDtypeStruct((NPAD, D), _f32))


def _em_weights(Wm, bm):
    wcat = jnp.concatenate([Wm[:D] - Wm[D:2 * D], Wm[D:2 * D]], axis=1)
    bcat = jnp.concatenate([jnp.zeros((D,), _f32), bm])[None, :]
    wea = Wm[2 * D]
    return wcat, bcat, wea


def _ec_weights(Wc, bc):
    wcat = jnp.concatenate([Wc[:D] - Wc[D:], Wc[D:]], axis=1)
    bcat = jnp.concatenate([bc, jnp.zeros((D,), _f32)])[None, :]
    return wcat, bcat


def kernel(x, edge_index, edge_attr, W1, b1, Wm1, bm1, W2, b2, Wm2, bm2,
           Wc1, bc1, Wc2, bc2, Wc3, bc3, Wc4, bc4, Wo, bo):
    ei0 = edge_index[0]
    ei1 = edge_index[1]

    p0 = jnp.argsort(ei0)
    key0 = ei0[p0]
    oth0 = ei1[p0]
    ea0 = edge_attr[:, 0][p0]
    p1 = jnp.argsort(ei1)
    dst1 = ei1[p1]
    src1 = ei0[p1]

    marks = jnp.arange(NW + 1, dtype=jnp.int32) * NPW
    ws0 = jnp.searchsorted(key0, marks).astype(jnp.int32)
    ws0 = jnp.concatenate([ws0, jnp.zeros((48 - NW - 1,), jnp.int32)])
    ws1 = jnp.searchsorted(dst1, marks).astype(jnp.int32)
    ws1 = jnp.concatenate([ws1, jnp.zeros((48 - NW - 1,), jnp.int32)])

    x_p = jnp.concatenate([x, jnp.zeros((NPAD - N, D), _f32)], axis=0)

    wm1cat, bm1cat, wea1 = _em_weights(Wm1, bm1)
    wm2cat, bm2cat, wea2 = _em_weights(Wm2, bm2)
    wc1cat, bc1cat = _ec_weights(Wc1, bc1)
    wc2cat, bc2cat = _ec_weights(Wc2, bc2)
    wc3cat, bc3cat = _ec_weights(Wc3, bc3)
    wc4cat, bc4cat = _ec_weights(Wc4, bc4)
    wo_p = jnp.zeros((D, D), _f32).at[:, :OUT].set(Wo)
    bo_p = jnp.zeros((1, D), _f32).at[0, :OUT].set(bo)

    h0, c1, b1m = _tc_head(x_p, W1, b1[None, :], wm1cat, bm1cat)
    agg1 = _emsum(c1, b1m, oth0, key0, ea0, wea1, ws0)

    h2, c2, b2m = _tc_em_mid(h0, agg1, W2, b2[None, :], wm2cat, bm2cat)
    agg2 = _emsum(c2, b2m, oth0, key0, ea0, wea2, ws0)

    c3, b3m = _tc_em2ec(h2, agg2, wc1cat, bc1cat)
    m3 = _edgemax(b3m, src1, dst1, ws1)

    c4, b4m = _tc_ec_mid(c3, m3, wc2cat, bc2cat)
    m4 = _edgemax(b4m, src1, dst1, ws1)

    c5, b5m = _tc_ec_mid(c4, m4, wc3cat, bc3cat)
    m5 = _edgemax(b5m, src1, dst1, ws1)

    c6, b6m = _tc_ec_mid(c5, m5, wc4cat, bc4cat)
    m6 = _edgemax(b6m, src1, dst1, ws1)

    y = _tc_out(c6, m6, wo_p, bo_p)
    return y[:N, :OUT]


# speedup vs baseline: 7.2601x; 2.1594x over previous
import functools

import jax
import jax.numpy as jnp
from jax import lax
from jax.experimental import pallas as pl
from jax.experimental.pallas import tpu as pltpu
from jax.experimental.pallas import tpu_sc as plsc

N = 10000
E = 320000
D = 128
OUT = 10

NW = 32
NC = 2
NPW = 320
NPAD = NW * NPW
W = 256
NEG_INF = float("-inf")

_mesh = plsc.VectorSubcoreMesh(core_axis_name="c", subcore_axis_name="s")


def _wid():
    return lax.axis_index("s") * NC + lax.axis_index("c")


NEG_BIG = -3.0e38


def _edgemax_body(b_hbm, src_hbm, dst_hbm, ws_hbm, m_hbm,
                  ws_v, idx0, idx1, dst0, dst1, rows0, rows1, mloc, spill,
                  sem0, sem1):
    wid = _wid()
    n0 = wid * NPW
    pltpu.sync_copy(ws_hbm, ws_v)
    e_lo = ws_v[pl.ds(wid, 16)][0]
    e_hi = ws_v[pl.ds(wid + 1, 16)][0]

    neg = jnp.full((16,), NEG_BIG, jnp.float32)

    def ibody(r, _):
        for v in range(8):
            mloc[r, pl.ds(v * 16, 16)] = neg
        return 0

    lax.fori_loop(0, NPW, ibody, 0)

    a_lo = (e_lo // 8) * 8
    nwin = (e_hi - a_lo + W - 1) // W
    npairs = jnp.maximum(nwin + 1, 2) // 2

    idxs = (idx0, idx1)
    dsts = (dst0, dst1)
    rows = (rows0, rows1)
    sems = (sem0, sem1)

    def win_base(k):
        start = a_lo + k * W
        return jnp.minimum(start, E - W), jnp.maximum(e_lo, start)

    def copy_idx(k, b):
        base, _ = win_base(k)
        pltpu.sync_copy(dst_hbm.at[pl.ds(base, W)], dsts[b])
        for j in range(2):
            pltpu.sync_copy(src_hbm.at[pl.ds(base + j * 128, 128)],
                            idxs[b].at[j])

    def fire(b):
        for j in range(2):
            pltpu.async_copy(b_hbm.at[idxs[b].at[j]],
                             rows[b].at[pl.ds(j * 128, 128)], sems[b])

    def drain(b):
        for j in range(2):
            pltpu.make_async_copy(b_hbm.at[idxs[b].at[j]],
                                  rows[b].at[pl.ds(j * 128, 128)],
                                  sems[b]).wait()

    for b in range(2):
        copy_idx(b, b)
        fire(b)

    for v in range(8):
        spill[pl.ds(v * 16, 16)] = neg

    def do_window(k, b, dprev):
        base, lo_k = win_base(k)
        drain(b)

        def gbody(g, dprev):
            acc = [spill[pl.ds(v * 16, 16)] for v in range(8)]
            off16 = g * 16
            dst16 = dsts[b][pl.ds(off16, 16)]
            for lane in range(16):
                i = off16 + lane
                e = base + i
                valid = jnp.logical_and(e >= lo_k, e < e_hi)
                vi = valid.astype(jnp.int32)
                d_eff = dprev + (dst16[lane] - dprev) * vi
                boundary = d_eff != dprev
                nbp = (dprev >= 0).astype(jnp.int32)
                flush_row = NPW + (dprev - n0 - NPW) * nbp
                s_inv = (1.0 - vi.astype(jnp.float32)) * NEG_BIG
                t_bnd = boundary.astype(jnp.float32) * NEG_BIG

                @pl.when(boundary)
                def _():
                    for v in range(8):
                        mloc[flush_row, pl.ds(v * 16, 16)] = acc[v]

                for v in range(8):
                    row = rows[b][i, pl.ds(v * 16, 16)] + s_inv
                    acc[v] = jnp.maximum(acc[v] + t_bnd, row)
                dprev = d_eff
            for v in range(8):
                spill[pl.ds(v * 16, 16)] = acc[v]
            return dprev

        dprev = lax.fori_loop(0, W // 16, gbody, dprev)
        copy_idx(k + 2, b)
        fire(b)
        return dprev

    def pbody(p, dprev):
        for b in range(2):
            dprev = do_window(2 * p + b, b, dprev)
        return dprev

    dprev = lax.fori_loop(0, npairs, pbody, jnp.int32(-1))

    flush_row = jnp.where(dprev >= 0, dprev - n0, NPW)
    for v in range(8):
        mloc[flush_row, pl.ds(v * 16, 16)] = spill[pl.ds(v * 16, 16)]
    for b in range(2):
        drain(b)

    pltpu.sync_copy(mloc.at[pl.ds(0, NPW)], m_hbm.at[pl.ds(n0, NPW)])


_edgemax = pl.kernel(
    _edgemax_body,
    out_type=jax.ShapeDtypeStruct((NPAD, D), jnp.float32),
    mesh=_mesh,
    scratch_types=[
        pltpu.VMEM((48,), jnp.int32),
        pltpu.VMEM((2, 128), jnp.int32),
        pltpu.VMEM((2, 128), jnp.int32),
        pltpu.VMEM((W,), jnp.int32),
        pltpu.VMEM((W,), jnp.int32),
        pltpu.VMEM((W, D), jnp.float32),
        pltpu.VMEM((W, D), jnp.float32),
        pltpu.VMEM((NPW + 1, D), jnp.float32),
        pltpu.VMEM((D,), jnp.float32),
        pltpu.SemaphoreType.DMA,
        pltpu.SemaphoreType.DMA,
    ],
)


def _emsum_body(c_hbm, b_hbm, oth_hbm, key_hbm, ea_hbm, w_hbm, ws_hbm, agg_hbm,
                ws_v, idx0, idx1, key0_v, key1_v, ea0_v, ea1_v, w_v,
                rows0, rows1, bloc, aggloc, spill_a, sem0, sem1):
    WS = 128
    wid = _wid()
    n0 = wid * NPW
    pltpu.sync_copy(ws_hbm, ws_v)
    pltpu.sync_copy(w_hbm, w_v)
    pltpu.sync_copy(b_hbm.at[pl.ds(n0, NPW)], bloc)
    e_lo = ws_v[pl.ds(wid, 16)][0]
    e_hi = ws_v[pl.ds(wid + 1, 16)][0]

    zero = jnp.zeros((16,), jnp.float32)

    def ibody(r, _):
        for v in range(8):
            aggloc[r, pl.ds(v * 16, 16)] = zero
        return 0

    lax.fori_loop(0, NPW, ibody, 0)

    wvec = [w_v[pl.ds(v * 16, 16)] for v in range(8)]

    a_lo = (e_lo // 8) * 8
    nwin = (e_hi - a_lo + WS - 1) // WS
    npairs = jnp.maximum(nwin + 1, 2) // 2

    idxs = (idx0, idx1)
    keys = (key0_v, key1_v)
    eas = (ea0_v, ea1_v)
    rows = (rows0, rows1)
    sems = (sem0, sem1)

    def win_base(k):
        start = a_lo + k * WS
        return jnp.minimum(start, E - WS), jnp.maximum(e_lo, start)

    def copy_idx(k, b):
        base, _ = win_base(k)
        pltpu.sync_copy(key_hbm.at[pl.ds(base, WS)], keys[b])
        pltpu.sync_copy(ea_hbm.at[pl.ds(base, WS)], eas[b])
        pltpu.sync_copy(oth_hbm.at[pl.ds(base, WS)], idxs[b])

    def fire(b):
        pltpu.async_copy(c_hbm.at[idxs[b]], rows[b], sems[b])

    def drain(b):
        pltpu.make_async_copy(c_hbm.at[idxs[b]], rows[b], sems[b]).wait()

    for b in range(2):
        copy_idx(b, b)
        fire(b)

    for v in range(8):
        spill_a[pl.ds(v * 16, 16)] = zero

    def do_window(k, b, kprev):
        base, lo_k = win_base(k)
        drain(b)

        def gbody(g, kprev):
            acc = [spill_a[pl.ds(v * 16, 16)] for v in range(8)]
            off16 = g * 16
            key16 = keys[b][pl.ds(off16, 16)]
            ea16 = eas[b][pl.ds(off16, 16)]
            for lane in range(16):
                i = off16 + lane
                e = base + i
                valid = jnp.logical_and(e >= lo_k, e < e_hi)
                vi = valid.astype(jnp.int32)
                d_eff = kprev + (key16[lane] - kprev) * vi
                boundary = d_eff != kprev
                nbp = (kprev >= 0).astype(jnp.int32)
                flush_row = NPW + (kprev - n0 - NPW) * nbp
                rcur = jnp.maximum(d_eff - n0, 0)
                ea = ea16[lane]
                vf = vi.astype(jnp.float32)
                bf = 1.0 - boundary.astype(jnp.float32)

                @pl.when(boundary)
                def _():
                    for v in range(8):
                        aggloc[flush_row, pl.ds(v * 16, 16)] = acc[v]

                for v in range(8):
                    c = rows[b][i, pl.ds(v * 16, 16)]
                    bbv = bloc[rcur, pl.ds(v * 16, 16)]
                    val = jnp.maximum(c + bbv + ea * wvec[v], 0.0) * vf
                    acc[v] = acc[v] * bf + val
                kprev = d_eff
            for v in range(8):
                spill_a[pl.ds(v * 16, 16)] = acc[v]
            return kprev

        kprev = lax.fori_loop(0, WS // 16, gbody, kprev)
        copy_idx(k + 2, b)
        fire(b)
        return kprev

    def pbody(p, kprev):
        for b in range(2):
            kprev = do_window(2 * p + b, b, kprev)
        return kprev

    kprev = lax.fori_loop(0, npairs, pbody, jnp.int32(-1))

    flush_row = jnp.where(kprev >= 0, kprev - n0, NPW)
    for v in range(8):
        aggloc[flush_row, pl.ds(v * 16, 16)] = spill_a[pl.ds(v * 16, 16)]
    for b in range(2):
        drain(b)

    pltpu.sync_copy(aggloc.at[pl.ds(0, NPW)], agg_hbm.at[pl.ds(n0, NPW)])


_emsum = pl.kernel(
    _emsum_body,
    out_type=jax.ShapeDtypeStruct((NPAD, D), jnp.float32),
    mesh=_mesh,
    scratch_types=[
        pltpu.VMEM((48,), jnp.int32),
        pltpu.VMEM((128,), jnp.int32),
        pltpu.VMEM((128,), jnp.int32),
        pltpu.VMEM((128,), jnp.int32),
        pltpu.VMEM((128,), jnp.int32),
        pltpu.VMEM((128,), jnp.float32),
        pltpu.VMEM((128,), jnp.float32),
        pltpu.VMEM((D,), jnp.float32),
        pltpu.VMEM((128, D), jnp.float32),
        pltpu.VMEM((128, D), jnp.float32),
        pltpu.VMEM((NPW, D), jnp.float32),
        pltpu.VMEM((NPW + 1, D), jnp.float32),
        pltpu.VMEM((D,), jnp.float32),
        pltpu.SemaphoreType.DMA,
        pltpu.SemaphoreType.DMA,
    ],
)


def _dot(a, b):
    return jnp.dot(a, b, preferred_element_type=jnp.float32,
                   precision=jax.lax.Precision.HIGHEST)


def _dot_ref(a, b):
    return jnp.dot(a, b, preferred_element_type=jnp.float32)


def _tc_head_body(x_ref, w1_ref, b1_ref, wcat_ref, bcat_ref, h_ref, c_ref, b_ref):
    h = jnp.maximum(_dot_ref(x_ref[...], w1_ref[...]) + b1_ref[...], 0.0)
    h_ref[...] = h
    cb = _dot(h, wcat_ref[...]) + bcat_ref[...]
    c_ref[...] = cb[:, :D]
    b_ref[...] = cb[:, D:]


def _tc_em_mid_body(h_ref_in, agg_ref, w2_ref, b2_ref, wcat_ref, bcat_ref,
                    h_ref, c_ref, b_ref):
    h1 = (h_ref_in[...] + agg_ref[...]) * 0.5
    h2 = jnp.maximum(_dot_ref(h1, w2_ref[...]) + b2_ref[...], 0.0)
    h_ref[...] = h2
    cb = _dot(h2, wcat_ref[...]) + bcat_ref[...]
    c_ref[...] = cb[:, :D]
    b_ref[...] = cb[:, D:]


def _tc_em2ec_body(h_ref_in, agg_ref, wcat_ref, bcat_ref, c_ref, b_ref):
    h3 = (h_ref_in[...] + agg_ref[...]) * 0.5
    cb = _dot(h3, wcat_ref[...]) + bcat_ref[...]
    c_ref[...] = cb[:, :D]
    b_ref[...] = cb[:, D:]


def _tc_ec_mid_body(cprev_ref, m_ref, wcat_ref, bcat_ref, c_ref, b_ref):
    m = m_ref[...]
    h = jnp.where(m < -1e38, 0.0, jnp.maximum(cprev_ref[...] + m, 0.0))
    cb = _dot(h, wcat_ref[...]) + bcat_ref[...]
    c_ref[...] = cb[:, :D]
    b_ref[...] = cb[:, D:]


def _tc_out_body(cprev_ref, m_ref, wo_ref, bo_ref, y_ref):
    m = m_ref[...]
    h = jnp.where(m < -1e38, 0.0, jnp.maximum(cprev_ref[...] + m, 0.0))
    y_ref[...] = _dot_ref(h, wo_ref[...]) + bo_ref[...]


_f32 = jnp.float32
_nd = jax.ShapeDtypeStruct((NPAD, D), _f32)
_BM = 2048
_G = NPAD // _BM


def _rspec():
    return pl.BlockSpec((_BM, D), lambda i: (i, 0))


def _wspec(shape):
    return pl.BlockSpec(shape, lambda i: (0, 0))


_tc_head = pl.pallas_call(
    _tc_head_body, grid=(_G,),
    in_specs=[_rspec(), _wspec((D, D)), _wspec((1, D)),
              _wspec((D, 2 * D)), _wspec((1, 2 * D))],
    out_specs=[_rspec(), _rspec(), _rspec()],
    out_shape=[_nd, _nd, _nd])
_tc_em_mid = pl.pallas_call(
    _tc_em_mid_body, grid=(_G,),
    in_specs=[_rspec(), _rspec(), _wspec((D, D)), _wspec((1, D)),
              _wspec((D, 2 * D)), _wspec((1, 2 * D))],
    out_specs=[_rspec(), _rspec(), _rspec()],
    out_shape=[_nd, _nd, _nd])
_tc_em2ec = pl.pallas_call(
    _tc_em2ec_body, grid=(_G,),
    in_specs=[_rspec(), _rspec(), _wspec((D, 2 * D)), _wspec((1, 2 * D))],
    out_specs=[_rspec(), _rspec()],
    out_shape=[_nd, _nd])
_tc_ec_mid = pl.pallas_call(
    _tc_ec_mid_body, grid=(_G,),
    in_specs=[_rspec(), _rspec(), _wspec((D, 2 * D)), _wspec((1, 2 * D))],
    out_specs=[_rspec(), _rspec()],
    out_shape=[_nd, _nd])
_tc_out = pl.pallas_call(
    _tc_out_body, grid=(_G,),
    in_specs=[_rspec(), _rspec(), _wspec((D, D)), _wspec((1, D))],
    out_specs=_rspec(),
    out_shape=jax.ShapeDtypeStruct((NPAD, D), _f32))


def _em_weights(Wm, bm):
    wcat = jnp.concatenate([Wm[:D] - Wm[D:2 * D], Wm[D:2 * D]], axis=1)
    bcat = jnp.concatenate([jnp.zeros((D,), _f32), bm])[None, :]
    wea = Wm[2 * D]
    return wcat, bcat, wea


def _ec_weights(Wc, bc):
    wcat = jnp.concatenate([Wc[:D] - Wc[D:], Wc[D:]], axis=1)
    bcat = jnp.concatenate([bc, jnp.zeros((D,), _f32)])[None, :]
    return wcat, bcat


def kernel(x, edge_index, edge_attr, W1, b1, Wm1, bm1, W2, b2, Wm2, bm2,
           Wc1, bc1, Wc2, bc2, Wc3, bc3, Wc4, bc4, Wo, bo):
    ei0 = edge_index[0]
    ei1 = edge_index[1]

    p0 = jnp.argsort(ei0)
    key0 = ei0[p0]
    oth0 = ei1[p0]
    ea0 = edge_attr[:, 0][p0]
    p1 = jnp.argsort(ei1)
    dst1 = ei1[p1]
    src1 = ei0[p1]

    marks = jnp.arange(NW + 1, dtype=jnp.int32) * NPW
    ws0 = jnp.searchsorted(key0, marks).astype(jnp.int32)
    ws0 = jnp.concatenate([ws0, jnp.zeros((48 - NW - 1,), jnp.int32)])
    ws1 = jnp.searchsorted(dst1, marks).astype(jnp.int32)
    ws1 = jnp.concatenate([ws1, jnp.zeros((48 - NW - 1,), jnp.int32)])

    x_p = jnp.concatenate([x, jnp.zeros((NPAD - N, D), _f32)], axis=0)

    wm1cat, bm1cat, wea1 = _em_weights(Wm1, bm1)
    wm2cat, bm2cat, wea2 = _em_weights(Wm2, bm2)
    wc1cat, bc1cat = _ec_weights(Wc1, bc1)
    wc2cat, bc2cat = _ec_weights(Wc2, bc2)
    wc3cat, bc3cat = _ec_weights(Wc3, bc3)
    wc4cat, bc4cat = _ec_weights(Wc4, bc4)
    wo_p = jnp.zeros((D, D), _f32).at[:, :OUT].set(Wo)
    bo_p = jnp.zeros((1, D), _f32).at[0, :OUT].set(bo)

    h0, c1, b1m = _tc_head(x_p, W1, b1[None, :], wm1cat, bm1cat)
    agg1 = _emsum(c1, b1m, oth0, key0, ea0, wea1, ws0)

    h2, c2, b2m = _tc_em_mid(h0, agg1, W2, b2[None, :], wm2cat, bm2cat)
    agg2 = _emsum(c2, b2m, oth0, key0, ea0, wea2, ws0)

    c3, b3m = _tc_em2ec(h2, agg2, wc1cat, bc1cat)
    m3 = _edgemax(b3m, src1, dst1, ws1)

    c4, b4m = _tc_ec_mid(c3, m3, wc2cat, bc2cat)
    m4 = _edgemax(b4m, src1, dst1, ws1)

    c5, b5m = _tc_ec_mid(c4, m4, wc3cat, bc3cat)
    m5 = _edgemax(b5m, src1, dst1, ws1)

    c6, b6m = _tc_ec_mid(c5, m5, wc4cat, bc4cat)
    m6 = _edgemax(b6m, src1, dst1, ws1)

    y = _tc_out(c6, m6, wo_p, bo_p)
    return y[:N, :OUT]


# speedup vs baseline: 8.7824x; 1.2097x over previous
import functools

import jax
import jax.numpy as jnp
from jax import lax
from jax.experimental import pallas as pl
from jax.experimental.pallas import tpu as pltpu
from jax.experimental.pallas import tpu_sc as plsc

N = 10000
E = 320000
D = 128
OUT = 10

NW = 32
NC = 2
NPW = 320
NPAD = NW * NPW
W = 256
NEG_INF = float("-inf")

_mesh = plsc.VectorSubcoreMesh(core_axis_name="c", subcore_axis_name="s")


def _wid():
    return lax.axis_index("s") * NC + lax.axis_index("c")


NEG_BIG = -3.0e38


def _edgemax_body(b_hbm, src_hbm, dst_hbm, ws_hbm, m_hbm,
                  ws_v, idx0, idx1, dst0, dst1, rows0, rows1, mloc, spill,
                  sem0, sem1, asem0, asem1, isem0, isem1):
    wid = _wid()
    n0 = wid * NPW
    pltpu.sync_copy(ws_hbm, ws_v)
    e_lo = ws_v[pl.ds(wid, 16)][0]
    e_hi = ws_v[pl.ds(wid + 1, 16)][0]

    neg = jnp.full((16,), NEG_BIG, jnp.float32)

    def ibody(r, _):
        for v in range(8):
            mloc[r, pl.ds(v * 16, 16)] = neg
        return 0

    lax.fori_loop(0, NPW, ibody, 0)

    a_lo = (e_lo // 8) * 8
    nwin = (e_hi - a_lo + W - 1) // W
    npairs = jnp.maximum(nwin + 1, 2) // 2

    idxs = (idx0, idx1)
    dsts = (dst0, dst1)
    rows = (rows0, rows1)
    sems = (sem0, sem1)
    asems = (asem0, asem1)
    isems = (isem0, isem1)

    def win_base(k):
        start = a_lo + k * W
        return jnp.minimum(start, E - W), jnp.maximum(e_lo, start)

    def fire_aux(k, b):
        base, _ = win_base(k)
        pltpu.async_copy(dst_hbm.at[pl.ds(base, W)], dsts[b], asems[b])

    def drain_aux(b):
        pltpu.make_async_copy(dst_hbm.at[pl.ds(0, W)], dsts[b],
                              asems[b]).wait()

    def fire_idx(k, b):
        base, _ = win_base(k)
        for j in range(2):
            pltpu.async_copy(src_hbm.at[pl.ds(base + j * 128, 128)],
                             idxs[b].at[j], isems[b])

    def drain_idx(b):
        for j in range(2):
            pltpu.make_async_copy(src_hbm.at[pl.ds(0, 128)], idxs[b].at[j],
                                  isems[b]).wait()

    def fire(b):
        for j in range(2):
            pltpu.async_copy(b_hbm.at[idxs[b].at[j]],
                             rows[b].at[pl.ds(j * 128, 128)], sems[b])

    def drain(b):
        for j in range(2):
            pltpu.make_async_copy(b_hbm.at[idxs[b].at[j]],
                                  rows[b].at[pl.ds(j * 128, 128)],
                                  sems[b]).wait()

    for b in range(2):
        fire_aux(b, b)
        fire_idx(b, b)
    for b in range(2):
        drain_idx(b)
        fire(b)

    for v in range(8):
        spill[pl.ds(v * 16, 16)] = neg

    def do_window(k, b, dprev):
        base, lo_k = win_base(k)
        drain(b)
        fire_idx(k + 2, b)
        drain_aux(b)

        def gbody(g, dprev):
            acc = [spill[pl.ds(v * 16, 16)] for v in range(8)]
            off16 = g * 16
            dst16 = dsts[b][pl.ds(off16, 16)]
            for lane in range(16):
                i = off16 + lane
                e = base + i
                valid = jnp.logical_and(e >= lo_k, e < e_hi)
                vi = valid.astype(jnp.int32)
                d_eff = dprev + (dst16[lane] - dprev) * vi
                boundary = d_eff != dprev
                nbp = (dprev >= 0).astype(jnp.int32)
                flush_row = NPW + (dprev - n0 - NPW) * nbp
                s_inv = (1.0 - vi.astype(jnp.float32)) * NEG_BIG
                t_bnd = boundary.astype(jnp.float32) * NEG_BIG

                @pl.when(boundary)
                def _():
                    for v in range(8):
                        mloc[flush_row, pl.ds(v * 16, 16)] = acc[v]

                for v in range(8):
                    row = rows[b][i, pl.ds(v * 16, 16)] + s_inv
                    acc[v] = jnp.maximum(acc[v] + t_bnd, row)
                dprev = d_eff
            for v in range(8):
                spill[pl.ds(v * 16, 16)] = acc[v]
            return dprev

        dprev = lax.fori_loop(0, W // 16, gbody, dprev)
        fire_aux(k + 2, b)
        drain_idx(b)
        fire(b)
        return dprev

    def pbody(p, dprev):
        for b in range(2):
            dprev = do_window(2 * p + b, b, dprev)
        return dprev

    dprev = lax.fori_loop(0, npairs, pbody, jnp.int32(-1))

    flush_row = jnp.where(dprev >= 0, dprev - n0, NPW)
    for v in range(8):
        mloc[flush_row, pl.ds(v * 16, 16)] = spill[pl.ds(v * 16, 16)]
    for b in range(2):
        drain_aux(b)
        drain(b)

    pltpu.sync_copy(mloc.at[pl.ds(0, NPW)], m_hbm.at[pl.ds(n0, NPW)])


_edgemax = pl.kernel(
    _edgemax_body,
    out_type=jax.ShapeDtypeStruct((NPAD, D), jnp.float32),
    mesh=_mesh,
    scratch_types=[
        pltpu.VMEM((48,), jnp.int32),
        pltpu.VMEM((2, 128), jnp.int32),
        pltpu.VMEM((2, 128), jnp.int32),
        pltpu.VMEM((W,), jnp.int32),
        pltpu.VMEM((W,), jnp.int32),
        pltpu.VMEM((W, D), jnp.float32),
        pltpu.VMEM((W, D), jnp.float32),
        pltpu.VMEM((NPW + 1, D), jnp.float32),
        pltpu.VMEM((D,), jnp.float32),
        pltpu.SemaphoreType.DMA,
        pltpu.SemaphoreType.DMA,
        pltpu.SemaphoreType.DMA,
        pltpu.SemaphoreType.DMA,
        pltpu.SemaphoreType.DMA,
        pltpu.SemaphoreType.DMA,
    ],
)


def _emsum_body(c_hbm, b_hbm, oth_hbm, key_hbm, ea_hbm, w_hbm, ws_hbm, agg_hbm,
                ws_v, idx0, idx1, key0_v, key1_v, ea0_v, ea1_v, w_v,
                rows0, rows1, bloc, aggloc, spill_a,
                sem0, sem1, asem0, asem1, isem0, isem1):
    WS = 128
    wid = _wid()
    n0 = wid * NPW
    pltpu.sync_copy(ws_hbm, ws_v)
    pltpu.sync_copy(w_hbm, w_v)
    pltpu.sync_copy(b_hbm.at[pl.ds(n0, NPW)], bloc)
    e_lo = ws_v[pl.ds(wid, 16)][0]
    e_hi = ws_v[pl.ds(wid + 1, 16)][0]

    zero = jnp.zeros((16,), jnp.float32)

    def ibody(r, _):
        for v in range(8):
            aggloc[r, pl.ds(v * 16, 16)] = zero
        return 0

    lax.fori_loop(0, NPW, ibody, 0)

    wvec = [w_v[pl.ds(v * 16, 16)] for v in range(8)]

    a_lo = (e_lo // 8) * 8
    nwin = (e_hi - a_lo + WS - 1) // WS
    npairs = jnp.maximum(nwin + 1, 2) // 2

    idxs = (idx0, idx1)
    keys = (key0_v, key1_v)
    eas = (ea0_v, ea1_v)
    rows = (rows0, rows1)
    sems = (sem0, sem1)
    asems = (asem0, asem1)
    isems = (isem0, isem1)

    def win_base(k):
        start = a_lo + k * WS
        return jnp.minimum(start, E - WS), jnp.maximum(e_lo, start)

    def fire_aux(k, b):
        base, _ = win_base(k)
        pltpu.async_copy(key_hbm.at[pl.ds(base, WS)], keys[b], asems[b])
        pltpu.async_copy(ea_hbm.at[pl.ds(base, WS)], eas[b], asems[b])

    def drain_aux(b):
        pltpu.make_async_copy(key_hbm.at[pl.ds(0, WS)], keys[b],
                              asems[b]).wait()
        pltpu.make_async_copy(ea_hbm.at[pl.ds(0, WS)], eas[b],
                              asems[b]).wait()

    def fire_idx(k, b):
        base, _ = win_base(k)
        pltpu.async_copy(oth_hbm.at[pl.ds(base, WS)], idxs[b], isems[b])

    def drain_idx(b):
        pltpu.make_async_copy(oth_hbm.at[pl.ds(0, WS)], idxs[b],
                              isems[b]).wait()

    def fire(b):
        pltpu.async_copy(c_hbm.at[idxs[b]], rows[b], sems[b])

    def drain(b):
        pltpu.make_async_copy(c_hbm.at[idxs[b]], rows[b], sems[b]).wait()

    for b in range(2):
        fire_aux(b, b)
        fire_idx(b, b)
    for b in range(2):
        drain_idx(b)
        fire(b)

    for v in range(8):
        spill_a[pl.ds(v * 16, 16)] = zero

    def do_window(k, b, kprev):
        base, lo_k = win_base(k)
        drain(b)
        fire_idx(k + 2, b)
        drain_aux(b)

        def gbody(g, kprev):
            acc = [spill_a[pl.ds(v * 16, 16)] for v in range(8)]
            off16 = g * 16
            key16 = keys[b][pl.ds(off16, 16)]
            ea16 = eas[b][pl.ds(off16, 16)]
            for lane in range(16):
                i = off16 + lane
                e = base + i
                valid = jnp.logical_and(e >= lo_k, e < e_hi)
                vi = valid.astype(jnp.int32)
                d_eff = kprev + (key16[lane] - kprev) * vi
                boundary = d_eff != kprev
                nbp = (kprev >= 0).astype(jnp.int32)
                flush_row = NPW + (kprev - n0 - NPW) * nbp
                rcur = jnp.maximum(d_eff - n0, 0)
                ea = ea16[lane]
                vf = vi.astype(jnp.float32)
                bf = 1.0 - boundary.astype(jnp.float32)

                @pl.when(boundary)
                def _():
                    for v in range(8):
                        aggloc[flush_row, pl.ds(v * 16, 16)] = acc[v]

                for v in range(8):
                    c = rows[b][i, pl.ds(v * 16, 16)]
                    bbv = bloc[rcur, pl.ds(v * 16, 16)]
                    val = jnp.maximum(c + bbv + ea * wvec[v], 0.0) * vf
                    acc[v] = acc[v] * bf + val
                kprev = d_eff
            for v in range(8):
                spill_a[pl.ds(v * 16, 16)] = acc[v]
            return kprev

        kprev = lax.fori_loop(0, WS // 16, gbody, kprev)
        fire_aux(k + 2, b)
        drain_idx(b)
        fire(b)
        return kprev

    def pbody(p, kprev):
        for b in range(2):
            kprev = do_window(2 * p + b, b, kprev)
        return kprev

    kprev = lax.fori_loop(0, npairs, pbody, jnp.int32(-1))

    flush_row = jnp.where(kprev >= 0, kprev - n0, NPW)
    for v in range(8):
        aggloc[flush_row, pl.ds(v * 16, 16)] = spill_a[pl.ds(v * 16, 16)]
    for b in range(2):
        drain_aux(b)
        drain(b)

    pltpu.sync_copy(aggloc.at[pl.ds(0, NPW)], agg_hbm.at[pl.ds(n0, NPW)])


_emsum = pl.kernel(
    _emsum_body,
    out_type=jax.ShapeDtypeStruct((NPAD, D), jnp.float32),
    mesh=_mesh,
    scratch_types=[
        pltpu.VMEM((48,), jnp.int32),
        pltpu.VMEM((128,), jnp.int32),
        pltpu.VMEM((128,), jnp.int32),
        pltpu.VMEM((128,), jnp.int32),
        pltpu.VMEM((128,), jnp.int32),
        pltpu.VMEM((128,), jnp.float32),
        pltpu.VMEM((128,), jnp.float32),
        pltpu.VMEM((D,), jnp.float32),
        pltpu.VMEM((128, D), jnp.float32),
        pltpu.VMEM((128, D), jnp.float32),
        pltpu.VMEM((NPW, D), jnp.float32),
        pltpu.VMEM((NPW + 1, D), jnp.float32),
        pltpu.VMEM((D,), jnp.float32),
        pltpu.SemaphoreType.DMA,
        pltpu.SemaphoreType.DMA,
        pltpu.SemaphoreType.DMA,
        pltpu.SemaphoreType.DMA,
        pltpu.SemaphoreType.DMA,
        pltpu.SemaphoreType.DMA,
    ],
)


def _dot(a, b):
    return jnp.dot(a, b, preferred_element_type=jnp.float32,
                   precision=jax.lax.Precision.HIGHEST)


def _dot_ref(a, b):
    return jnp.dot(a, b, preferred_element_type=jnp.float32)


def _tc_head_body(x_ref, w1_ref, b1_ref, wcat_ref, bcat_ref, h_ref, c_ref, b_ref):
    h = jnp.maximum(_dot_ref(x_ref[...], w1_ref[...]) + b1_ref[...], 0.0)
    h_ref[...] = h
    cb = _dot(h, wcat_ref[...]) + bcat_ref[...]
    c_ref[...] = cb[:, :D]
    b_ref[...] = cb[:, D:]


def _tc_em_mid_body(h_ref_in, agg_ref, w2_ref, b2_ref, wcat_ref, bcat_ref,
                    h_ref, c_ref, b_ref):
    h1 = (h_ref_in[...] + agg_ref[...]) * 0.5
    h2 = jnp.maximum(_dot_ref(h1, w2_ref[...]) + b2_ref[...], 0.0)
    h_ref[...] = h2
    cb = _dot(h2, wcat_ref[...]) + bcat_ref[...]
    c_ref[...] = cb[:, :D]
    b_ref[...] = cb[:, D:]


def _tc_em2ec_body(h_ref_in, agg_ref, wcat_ref, bcat_ref, c_ref, b_ref):
    h3 = (h_ref_in[...] + agg_ref[...]) * 0.5
    cb = _dot(h3, wcat_ref[...]) + bcat_ref[...]
    c_ref[...] = cb[:, :D]
    b_ref[...] = cb[:, D:]


def _tc_ec_mid_body(cprev_ref, m_ref, wcat_ref, bcat_ref, c_ref, b_ref):
    m = m_ref[...]
    h = jnp.where(m < -1e38, 0.0, jnp.maximum(cprev_ref[...] + m, 0.0))
    cb = _dot(h, wcat_ref[...]) + bcat_ref[...]
    c_ref[...] = cb[:, :D]
    b_ref[...] = cb[:, D:]


def _tc_out_body(cprev_ref, m_ref, wo_ref, bo_ref, y_ref):
    m = m_ref[...]
    h = jnp.where(m < -1e38, 0.0, jnp.maximum(cprev_ref[...] + m, 0.0))
    y_ref[...] = _dot_ref(h, wo_ref[...]) + bo_ref[...]


_f32 = jnp.float32
_nd = jax.ShapeDtypeStruct((NPAD, D), _f32)
_BM = 2048
_G = NPAD // _BM


def _rspec():
    return pl.BlockSpec((_BM, D), lambda i: (i, 0))


def _wspec(shape):
    return pl.BlockSpec(shape, lambda i: (0, 0))


_tc_head = pl.pallas_call(
    _tc_head_body, grid=(_G,),
    in_specs=[_rspec(), _wspec((D, D)), _wspec((1, D)),
              _wspec((D, 2 * D)), _wspec((1, 2 * D))],
    out_specs=[_rspec(), _rspec(), _rspec()],
    out_shape=[_nd, _nd, _nd])
_tc_em_mid = pl.pallas_call(
    _tc_em_mid_body, grid=(_G,),
    in_specs=[_rspec(), _rspec(), _wspec((D, D)), _wspec((1, D)),
              _wspec((D, 2 * D)), _wspec((1, 2 * D))],
    out_specs=[_rspec(), _rspec(), _rspec()],
    out_shape=[_nd, _nd, _nd])
_tc_em2ec = pl.pallas_call(
    _tc_em2ec_body, grid=(_G,),
    in_specs=[_rspec(), _rspec(), _wspec((D, 2 * D)), _wspec((1, 2 * D))],
    out_specs=[_rspec(), _rspec()],
    out_shape=[_nd, _nd])
_tc_ec_mid = pl.pallas_call(
    _tc_ec_mid_body, grid=(_G,),
    in_specs=[_rspec(), _rspec(), _wspec((D, 2 * D)), _wspec((1, 2 * D))],
    out_specs=[_rspec(), _rspec()],
    out_shape=[_nd, _nd])
_tc_out = pl.pallas_call(
    _tc_out_body, grid=(_G,),
    in_specs=[_rspec(), _rspec(), _wspec((D, D)), _wspec((1, D))],
    out_specs=_rspec(),
    out_shape=jax.ShapeDtypeStruct((NPAD, D), _f32))


def _em_weights(Wm, bm):
    wcat = jnp.concatenate([Wm[:D] - Wm[D:2 * D], Wm[D:2 * D]], axis=1)
    bcat = jnp.concatenate([jnp.zeros((D,), _f32), bm])[None, :]
    wea = Wm[2 * D]
    return wcat, bcat, wea


def _ec_weights(Wc, bc):
    wcat = jnp.concatenate([Wc[:D] - Wc[D:], Wc[D:]], axis=1)
    bcat = jnp.concatenate([bc, jnp.zeros((D,), _f32)])[None, :]
    return wcat, bcat


def kernel(x, edge_index, edge_attr, W1, b1, Wm1, bm1, W2, b2, Wm2, bm2,
           Wc1, bc1, Wc2, bc2, Wc3, bc3, Wc4, bc4, Wo, bo):
    ei0 = edge_index[0]
    ei1 = edge_index[1]

    p0 = jnp.argsort(ei0)
    key0 = ei0[p0]
    oth0 = ei1[p0]
    ea0 = edge_attr[:, 0][p0]
    p1 = jnp.argsort(ei1)
    dst1 = ei1[p1]
    src1 = ei0[p1]

    marks = jnp.arange(NW + 1, dtype=jnp.int32) * NPW
    ws0 = jnp.searchsorted(key0, marks).astype(jnp.int32)
    ws0 = jnp.concatenate([ws0, jnp.zeros((48 - NW - 1,), jnp.int32)])
    ws1 = jnp.searchsorted(dst1, marks).astype(jnp.int32)
    ws1 = jnp.concatenate([ws1, jnp.zeros((48 - NW - 1,), jnp.int32)])

    x_p = jnp.concatenate([x, jnp.zeros((NPAD - N, D), _f32)], axis=0)

    wm1cat, bm1cat, wea1 = _em_weights(Wm1, bm1)
    wm2cat, bm2cat, wea2 = _em_weights(Wm2, bm2)
    wc1cat, bc1cat = _ec_weights(Wc1, bc1)
    wc2cat, bc2cat = _ec_weights(Wc2, bc2)
    wc3cat, bc3cat = _ec_weights(Wc3, bc3)
    wc4cat, bc4cat = _ec_weights(Wc4, bc4)
    wo_p = jnp.zeros((D, D), _f32).at[:, :OUT].set(Wo)
    bo_p = jnp.zeros((1, D), _f32).at[0, :OUT].set(bo)

    h0, c1, b1m = _tc_head(x_p, W1, b1[None, :], wm1cat, bm1cat)
    agg1 = _emsum(c1, b1m, oth0, key0, ea0, wea1, ws0)

    h2, c2, b2m = _tc_em_mid(h0, agg1, W2, b2[None, :], wm2cat, bm2cat)
    agg2 = _emsum(c2, b2m, oth0, key0, ea0, wea2, ws0)

    c3, b3m = _tc_em2ec(h2, agg2, wc1cat, bc1cat)
    m3 = _edgemax(b3m, src1, dst1, ws1)

    c4, b4m = _tc_ec_mid(c3, m3, wc2cat, bc2cat)
    m4 = _edgemax(b4m, src1, dst1, ws1)

    c5, b5m = _tc_ec_mid(c4, m4, wc3cat, bc3cat)
    m5 = _edgemax(b5m, src1, dst1, ws1)

    c6, b6m = _tc_ec_mid(c5, m5, wc4cat, bc4cat)
    m6 = _edgemax(b6m, src1, dst1, ws1)

    y = _tc_out(c6, m6, wo_p, bo_p)
    return y[:N, :OUT]
